# Initial kernel scaffold; baseline (speedup 1.0000x reference)
import jax
import jax.numpy as jnp
from jax.experimental import pallas as pl


def kernel(x, edge_index, W1, b1, W2, b2, W3, b3):
    raise NotImplementedError("write your pallas kernel here")


# speedup vs baseline: 40.3362x; 40.3362x over previous
import functools

import jax
import jax.numpy as jnp
from jax import lax
from jax.experimental import pallas as pl
from jax.experimental.pallas import tpu as pltpu
from jax.experimental.pallas import tpu_sc as plsc

_SC_PARAMS = pltpu.CompilerParams(use_tc_tiling_on_sc=False)

_NC = 2
_NS = 16
_NW = _NC * _NS
_CHUNK = 128
_DEG_W = 16


@functools.lru_cache(maxsize=None)
def _sc_segment_sum(n_pad: int, cpt: int, d: int):
    rpt = n_pad // _NS
    mesh = plsc.VectorSubcoreMesh(core_axis_name="c", subcore_axis_name="s")

    @functools.partial(
        pl.kernel,
        out_type=jax.ShapeDtypeStruct((_NC, n_pad, d), jnp.float32),
        mesh=mesh,
        scratch_types=[
            pltpu.VMEM((cpt, _CHUNK), jnp.int32),
            pltpu.VMEM((cpt, _CHUNK), jnp.int32),
            pltpu.VMEM((4, _CHUNK, d), jnp.float32),
            pltpu.VMEM((64, d), jnp.float32),
            pltpu.VMEM_SHARED((n_pad, d), jnp.float32),
            pltpu.SemaphoreType.DMA((4,)),
            pltpu.SemaphoreType.DMA((4,)),
        ],
        compiler_params=_SC_PARAMS,
    )
    def kern(g_hbm, srcs_hbm, dsts_hbm, out_hbm,
             src_v, dst_v, rows_v, zero_v, acc_sh, sem_g, sem_s):
        cid = lax.axis_index("c")
        sid = lax.axis_index("s")
        w = cid * _NS + sid

        pltpu.sync_copy(srcs_hbm.at[w], src_v)
        pltpu.sync_copy(dsts_hbm.at[w], dst_v)

        zvec = jnp.zeros((16,), jnp.float32)

        @pl.loop(0, 64)
        def _(r):
            @pl.loop(0, d, step=16)
            def _(c0):
                zero_v[r, pl.ds(c0, 16)] = zvec

        base = sid * rpt

        @pl.loop(0, rpt, step=64)
        def _(r0):
            pltpu.sync_copy(zero_v, acc_sh.at[pl.ds(base + r0, 64)])

        plsc.subcore_barrier()

        pltpu.async_copy(g_hbm.at[src_v.at[0]], rows_v.at[0], sem_g.at[0])
        pltpu.async_copy(g_hbm.at[src_v.at[1]], rows_v.at[1], sem_g.at[1])

        @pl.loop(0, cpt, step=4)
        def _(k0):
            for u in range(4):
                k = k0 + u
                bn = (u + 2) % 4
                pltpu.make_async_copy(g_hbm.at[src_v.at[k]], rows_v.at[u],
                                      sem_g.at[u]).wait()
                pltpu.async_copy(rows_v.at[u], acc_sh.at[dst_v.at[k]],
                                 sem_s.at[u], add=True)

                @pl.when(k + 2 < cpt)
                def _():
                    @pl.when(k >= 2)
                    def _():
                        pltpu.make_async_copy(rows_v.at[bn],
                                              acc_sh.at[dst_v.at[k - 2]],
                                              sem_s.at[bn]).wait()
                    pltpu.async_copy(g_hbm.at[src_v.at[k + 2]], rows_v.at[bn],
                                     sem_g.at[bn])

        for u in range(4):
            pltpu.make_async_copy(rows_v.at[u], acc_sh.at[dst_v.at[cpt - 4 + u]],
                                  sem_s.at[u]).wait()

        plsc.subcore_barrier()
        pltpu.sync_copy(acc_sh.at[pl.ds(base, rpt)],
                        out_hbm.at[cid, pl.ds(base, rpt)])

    return kern


def _rsqrt_vec(v):
    i = lax.bitcast_convert_type(v, jnp.int32)
    y = lax.bitcast_convert_type(jnp.int32(0x5F3759DF) - (i >> 1), jnp.float32)
    y = y * (1.5 - 0.5 * v * y * y)
    y = y * (1.5 - 0.5 * v * y * y)
    y = y * (1.5 - 0.5 * v * y * y)
    return y


@functools.lru_cache(maxsize=None)
def _sc_degree_dinv(n_pad: int, cpt: int):
    d = _DEG_W
    rpt = n_pad // _NS
    half = rpt // 2
    mesh = plsc.VectorSubcoreMesh(core_axis_name="c", subcore_axis_name="s")

    @functools.partial(
        pl.kernel,
        out_type=jax.ShapeDtypeStruct((n_pad // 2, 128), jnp.float32),
        mesh=mesh,
        scratch_types=[
            pltpu.VMEM((2 * cpt, _CHUNK), jnp.int32),
            pltpu.VMEM((_CHUNK, d), jnp.float32),
            pltpu.VMEM((64, d), jnp.float32),
            pltpu.VMEM((rpt, d), jnp.float32),
            pltpu.VMEM((half, 128), jnp.float32),
            pltpu.VMEM_SHARED((n_pad, d), jnp.float32),
            pltpu.SemaphoreType.DMA((2,)),
        ],
        compiler_params=_SC_PARAMS,
    )
    def kern(dsts_hbm, out_hbm, dst_v, ones_v, zero_v, deg_v, dinv_v,
             acc_sh, sem_s):
        cid = lax.axis_index("c")
        sid = lax.axis_index("s")

        @pl.when(cid == 0)
        def _():
            pltpu.sync_copy(dsts_hbm.at[2 * sid], dst_v.at[pl.ds(0, cpt)])
            pltpu.sync_copy(dsts_hbm.at[2 * sid + 1], dst_v.at[pl.ds(cpt, cpt)])

            zvec = jnp.zeros((16,), jnp.float32)
            ovec = jnp.ones((16,), jnp.float32)

            @pl.loop(0, _CHUNK)
            def _(r):
                ones_v[r, pl.ds(0, 16)] = ovec

            @pl.loop(0, 64)
            def _(r):
                zero_v[r, pl.ds(0, 16)] = zvec

            base = sid * rpt

            @pl.loop(0, rpt, step=64)
            def _(r0):
                pltpu.sync_copy(zero_v, acc_sh.at[pl.ds(base + r0, 64)])

            plsc.subcore_barrier()

            @pl.loop(0, 2 * cpt, step=2)
            def _(j):
                for u in range(2):
                    k = j + u

                    @pl.when(k >= 2)
                    def _():
                        pltpu.make_async_copy(ones_v, acc_sh.at[dst_v.at[k - 2]],
                                              sem_s.at[u]).wait()

                    pltpu.async_copy(ones_v, acc_sh.at[dst_v.at[k]],
                                     sem_s.at[u], add=True)

            for u in range(2):
                pltpu.make_async_copy(ones_v, acc_sh.at[dst_v.at[2 * cpt - 2 + u]],
                                      sem_s.at[u]).wait()

            plsc.subcore_barrier()

            pltpu.sync_copy(acc_sh.at[pl.ds(base, rpt)], deg_v)

            @pl.loop(0, half)
            def _(r2):
                ye = _rsqrt_vec(deg_v[2 * r2, pl.ds(0, 16)] + 1.0)
                yo = _rsqrt_vec(deg_v[2 * r2 + 1, pl.ds(0, 16)] + 1.0)
                for q in range(4):
                    dinv_v[r2, pl.ds(16 * q, 16)] = ye
                for q in range(4, 8):
                    dinv_v[r2, pl.ds(16 * q, 16)] = yo

            pltpu.sync_copy(dinv_v, out_hbm.at[pl.ds(sid * half, half)])

    return kern


_MM = dict(preferred_element_type=jnp.float32, precision=lax.Precision.HIGHEST)
_R2 = 512


def _h1_body(x_ref, w_ref, o_ref):
    o_ref[...] = jnp.dot(x_ref[...], w_ref[...], **_MM)


def _scale_body(h_ref, d_ref, g_ref):
    g_ref[...] = d_ref[...] * h_ref[...]


def _layer_body(p_ref, g_ref, b_ref, w_ref, d_ref, o_ref):
    dinv = d_ref[...]
    s = p_ref[0] + p_ref[1] + g_ref[...]
    o = jnp.maximum(dinv * s + b_ref[...], 0.0)
    o_ref[...] = dinv * jnp.dot(o, w_ref[...], **_MM)


def _final_body(p_ref, g_ref, b_ref, d_ref, o_ref):
    o_ref[...] = d_ref[...] * (p_ref[0] + p_ref[1] + g_ref[...]) + b_ref[...]


def _tc_h1(xpair, W1_bd):
    n2 = xpair.shape[0]
    return pl.pallas_call(
        _h1_body,
        grid=(n2 // _R2,),
        in_specs=[
            pl.BlockSpec((_R2, 256), lambda i: (i, 0)),
            pl.BlockSpec((256, 128), lambda i: (0, 0)),
        ],
        out_specs=pl.BlockSpec((_R2, 128), lambda i: (i, 0)),
        out_shape=jax.ShapeDtypeStruct((n2, 128), jnp.float32),
    )(xpair, W1_bd)


def _tc_scale(hp, d128):
    n2 = hp.shape[0]
    return pl.pallas_call(
        _scale_body,
        grid=(n2 // _R2,),
        in_specs=[
            pl.BlockSpec((_R2, 128), lambda i: (i, 0)),
            pl.BlockSpec((_R2, 128), lambda i: (i, 0)),
        ],
        out_specs=pl.BlockSpec((_R2, 128), lambda i: (i, 0)),
        out_shape=jax.ShapeDtypeStruct((n2, 128), jnp.float32),
    )(hp, d128)


def _tc_layer(p, g, b, W_bd, d128):
    n2 = g.shape[0]
    return pl.pallas_call(
        _layer_body,
        grid=(n2 // _R2,),
        in_specs=[
            pl.BlockSpec((2, _R2, 128), lambda i: (0, i, 0)),
            pl.BlockSpec((_R2, 128), lambda i: (i, 0)),
            pl.BlockSpec((1, 128), lambda i: (0, 0)),
            pl.BlockSpec((128, 128), lambda i: (0, 0)),
            pl.BlockSpec((_R2, 128), lambda i: (i, 0)),
        ],
        out_specs=pl.BlockSpec((_R2, 128), lambda i: (i, 0)),
        out_shape=jax.ShapeDtypeStruct((n2, 128), jnp.float32),
    )(p, g, b, W_bd, d128)


def _tc_final(p, g, b, d128):
    n2 = g.shape[0]
    return pl.pallas_call(
        _final_body,
        grid=(n2 // _R2,),
        in_specs=[
            pl.BlockSpec((2, _R2, 128), lambda i: (0, i, 0)),
            pl.BlockSpec((_R2, 128), lambda i: (i, 0)),
            pl.BlockSpec((1, 128), lambda i: (0, 0)),
            pl.BlockSpec((_R2, 128), lambda i: (i, 0)),
        ],
        out_specs=pl.BlockSpec((_R2, 128), lambda i: (i, 0)),
        out_shape=jax.ShapeDtypeStruct((n2, 128), jnp.float32),
    )(p, g, b, d128)


def _blockdiag(W):
    r, c = W.shape
    return (jnp.zeros((2 * r, 2 * c), jnp.float32)
            .at[:r, :c].set(W).at[r:, c:].set(W))


def kernel(x, edge_index, W1, b1, W2, b2, W3, b3):
    n = x.shape[0]
    e = edge_index.shape[1]
    n_pad = -(-n // 1024) * 1024
    n2 = n_pad // 2
    cpt = -(-e // (_NW * _CHUNK))
    cpt = -(-cpt // 4) * 4
    e_pad = _NW * cpt * _CHUNK

    src = edge_index[0].astype(jnp.int32)
    dst = edge_index[1].astype(jnp.int32)
    spare = n_pad - n
    if spare > 0:
        fill = n + jnp.arange(e_pad - e, dtype=jnp.int32) % spare
    else:
        fill = jnp.full((e_pad - e,), n_pad - 1, jnp.int32)
    srcs = jnp.concatenate([src, fill]).reshape(_NW, cpt, _CHUNK)
    dsts = jnp.concatenate([dst, fill]).reshape(_NW, cpt, _CHUNK)

    xpair = (jnp.zeros((n2, 256), jnp.float32)
             .at[:n // 2].set(x.reshape(n // 2, 256)))
    W1_bd = _blockdiag(W1)
    W2_bd = _blockdiag(W2)
    W3_bd = _blockdiag(jnp.zeros((64, 64), jnp.float32).at[:, :6].set(W3))
    b1_bd = jnp.concatenate([b1, b1]).reshape(1, 128)
    b2_bd = jnp.concatenate([b2, b2]).reshape(1, 128)
    b3p = jnp.zeros((64,), jnp.float32).at[:6].set(b3)
    b3_bd = jnp.concatenate([b3p, b3p]).reshape(1, 128)

    d128 = _sc_degree_dinv(n_pad, cpt)(dsts)

    agg = _sc_segment_sum(n_pad, cpt, 64)
    h1p = _tc_h1(xpair, W1_bd)
    g1p = _tc_scale(h1p, d128)
    p1 = agg(g1p.reshape(n_pad, 64), srcs, dsts)
    g2p = _tc_layer(p1.reshape(2, n2, 128), g1p, b1_bd, W2_bd, d128)
    p2 = agg(g2p.reshape(n_pad, 64), srcs, dsts)
    g3p = _tc_layer(p2.reshape(2, n2, 128), g2p, b2_bd, W3_bd, d128)
    p3 = agg(g3p.reshape(n_pad, 64), srcs, dsts)
    outp = _tc_final(p3.reshape(2, n2, 128), g3p, b3_bd, d128)
    return outp.reshape(n_pad, 64)[:n, :6]


# speedup vs baseline: 43.7757x; 1.0853x over previous
import functools

import jax
import jax.numpy as jnp
from jax import lax
from jax.experimental import pallas as pl
from jax.experimental.pallas import tpu as pltpu
from jax.experimental.pallas import tpu_sc as plsc

_SC_PARAMS = pltpu.CompilerParams(use_tc_tiling_on_sc=False)

_NC = 2
_NS = 16
_NW = _NC * _NS
_CHUNK = 128
_DEG_W = 16


@functools.lru_cache(maxsize=None)
def _sc_segment_sum(n_pad: int, cpt: int, d: int):
    rpt = n_pad // _NS
    mesh = plsc.VectorSubcoreMesh(core_axis_name="c", subcore_axis_name="s")

    @functools.partial(
        pl.kernel,
        out_type=jax.ShapeDtypeStruct((_NC, n_pad, d), jnp.float32),
        mesh=mesh,
        scratch_types=[
            pltpu.VMEM((cpt, _CHUNK), jnp.int32),
            pltpu.VMEM((cpt, _CHUNK), jnp.int32),
            pltpu.VMEM((8, _CHUNK, d), jnp.float32),
            pltpu.VMEM((64, d), jnp.float32),
            pltpu.VMEM_SHARED((n_pad, d), jnp.float32),
            pltpu.SemaphoreType.DMA((8,)),
            pltpu.SemaphoreType.DMA((8,)),
        ],
        compiler_params=_SC_PARAMS,
    )
    def kern(g_hbm, ei_hbm, out_hbm,
             src_v, dst_v, rows_v, zero_v, acc_sh, sem_g, sem_s):
        cid = lax.axis_index("c")
        sid = lax.axis_index("s")
        w = cid * _NS + sid

        pltpu.sync_copy(ei_hbm.at[0, w], src_v)
        pltpu.sync_copy(ei_hbm.at[1, w], dst_v)

        zvec = jnp.zeros((16,), jnp.float32)

        @pl.loop(0, 64)
        def _(r):
            @pl.loop(0, d, step=16)
            def _(c0):
                zero_v[r, pl.ds(c0, 16)] = zvec

        base = sid * rpt

        @pl.loop(0, rpt, step=64)
        def _(r0):
            pltpu.sync_copy(zero_v, acc_sh.at[pl.ds(base + r0, 64)])

        plsc.subcore_barrier()

        for u in range(4):
            pltpu.async_copy(g_hbm.at[src_v.at[u]], rows_v.at[u], sem_g.at[u])

        @pl.loop(0, cpt, step=8)
        def _(k0):
            for u in range(8):
                k = k0 + u
                bn = (u + 4) % 8
                pltpu.make_async_copy(g_hbm.at[src_v.at[k]], rows_v.at[u],
                                      sem_g.at[u]).wait()
                pltpu.async_copy(rows_v.at[u], acc_sh.at[dst_v.at[k]],
                                 sem_s.at[u], add=True)

                @pl.when(k + 4 < cpt)
                def _():
                    @pl.when(k >= 4)
                    def _():
                        pltpu.make_async_copy(rows_v.at[bn],
                                              acc_sh.at[dst_v.at[k - 4]],
                                              sem_s.at[bn]).wait()
                    pltpu.async_copy(g_hbm.at[src_v.at[k + 4]], rows_v.at[bn],
                                     sem_g.at[bn])

        for u in range(8):
            pltpu.make_async_copy(rows_v.at[u], acc_sh.at[dst_v.at[cpt - 8 + u]],
                                  sem_s.at[u]).wait()

        plsc.subcore_barrier()
        pltpu.sync_copy(acc_sh.at[pl.ds(base, rpt)],
                        out_hbm.at[cid, pl.ds(base, rpt)])

    return kern


def _rsqrt_vec(v):
    i = lax.bitcast_convert_type(v, jnp.int32)
    y = lax.bitcast_convert_type(jnp.int32(0x5F3759DF) - (i >> 1), jnp.float32)
    y = y * (1.5 - 0.5 * v * y * y)
    y = y * (1.5 - 0.5 * v * y * y)
    y = y * (1.5 - 0.5 * v * y * y)
    return y


@functools.lru_cache(maxsize=None)
def _sc_degree_dinv(n_pad: int, cpt: int):
    d = _DEG_W
    rpt = n_pad // _NS
    half = rpt // 2
    mesh = plsc.VectorSubcoreMesh(core_axis_name="c", subcore_axis_name="s")

    @functools.partial(
        pl.kernel,
        out_type=jax.ShapeDtypeStruct((n_pad // 2, 128), jnp.float32),
        mesh=mesh,
        scratch_types=[
            pltpu.VMEM((2 * cpt, _CHUNK), jnp.int32),
            pltpu.VMEM((_CHUNK, d), jnp.float32),
            pltpu.VMEM((64, d), jnp.float32),
            pltpu.VMEM((rpt, d), jnp.float32),
            pltpu.VMEM((half, 128), jnp.float32),
            pltpu.VMEM_SHARED((n_pad, d), jnp.float32),
            pltpu.SemaphoreType.DMA((2,)),
        ],
        compiler_params=_SC_PARAMS,
    )
    def kern(ei_hbm, out_hbm, dst_v, ones_v, zero_v, deg_v, dinv_v,
             acc_sh, sem_s):
        cid = lax.axis_index("c")
        sid = lax.axis_index("s")

        @pl.when(cid == 0)
        def _():
            pltpu.sync_copy(ei_hbm.at[1, 2 * sid], dst_v.at[pl.ds(0, cpt)])
            pltpu.sync_copy(ei_hbm.at[1, 2 * sid + 1], dst_v.at[pl.ds(cpt, cpt)])

            zvec = jnp.zeros((16,), jnp.float32)
            ovec = jnp.ones((16,), jnp.float32)

            @pl.loop(0, _CHUNK)
            def _(r):
                ones_v[r, pl.ds(0, 16)] = ovec

            @pl.loop(0, 64)
            def _(r):
                zero_v[r, pl.ds(0, 16)] = zvec

            base = sid * rpt

            @pl.loop(0, rpt, step=64)
            def _(r0):
                pltpu.sync_copy(zero_v, acc_sh.at[pl.ds(base + r0, 64)])

            plsc.subcore_barrier()

            @pl.loop(0, 2 * cpt, step=2)
            def _(j):
                for u in range(2):
                    k = j + u

                    @pl.when(k >= 2)
                    def _():
                        pltpu.make_async_copy(ones_v, acc_sh.at[dst_v.at[k - 2]],
                                              sem_s.at[u]).wait()

                    pltpu.async_copy(ones_v, acc_sh.at[dst_v.at[k]],
                                     sem_s.at[u], add=True)

            for u in range(2):
                pltpu.make_async_copy(ones_v, acc_sh.at[dst_v.at[2 * cpt - 2 + u]],
                                      sem_s.at[u]).wait()

            plsc.subcore_barrier()

            pltpu.sync_copy(acc_sh.at[pl.ds(base, rpt)], deg_v)

            @pl.loop(0, half)
            def _(r2):
                ye = _rsqrt_vec(deg_v[2 * r2, pl.ds(0, 16)] + 1.0)
                yo = _rsqrt_vec(deg_v[2 * r2 + 1, pl.ds(0, 16)] + 1.0)
                for q in range(4):
                    dinv_v[r2, pl.ds(16 * q, 16)] = ye
                for q in range(4, 8):
                    dinv_v[r2, pl.ds(16 * q, 16)] = yo

            pltpu.sync_copy(dinv_v, out_hbm.at[pl.ds(sid * half, half)])

    return kern


_MM = dict(preferred_element_type=jnp.float32, precision=lax.Precision.HIGHEST)
_R2 = 512


def _h1_body(x_ref, w_ref, o_ref):
    o_ref[...] = jnp.dot(x_ref[...], w_ref[...], **_MM)


def _scale_body(h_ref, d_ref, g_ref):
    g_ref[...] = d_ref[...] * h_ref[...]


def _layer_body(p_ref, g_ref, b_ref, w_ref, d_ref, o_ref):
    dinv = d_ref[...]
    s = p_ref[0] + p_ref[1] + g_ref[...]
    o = jnp.maximum(dinv * s + b_ref[...], 0.0)
    o_ref[...] = dinv * jnp.dot(o, w_ref[...], **_MM)


def _final_body(p_ref, g_ref, b_ref, d_ref, o_ref):
    o_ref[...] = d_ref[...] * (p_ref[0] + p_ref[1] + g_ref[...]) + b_ref[...]


def _tc_h1(xpair, W1_bd):
    n2 = xpair.shape[0]
    return pl.pallas_call(
        _h1_body,
        grid=(n2 // _R2,),
        in_specs=[
            pl.BlockSpec((_R2, 256), lambda i: (i, 0)),
            pl.BlockSpec((256, 128), lambda i: (0, 0)),
        ],
        out_specs=pl.BlockSpec((_R2, 128), lambda i: (i, 0)),
        out_shape=jax.ShapeDtypeStruct((n2, 128), jnp.float32),
    )(xpair, W1_bd)


def _tc_scale(hp, d128):
    n2 = hp.shape[0]
    return pl.pallas_call(
        _scale_body,
        grid=(n2 // _R2,),
        in_specs=[
            pl.BlockSpec((_R2, 128), lambda i: (i, 0)),
            pl.BlockSpec((_R2, 128), lambda i: (i, 0)),
        ],
        out_specs=pl.BlockSpec((_R2, 128), lambda i: (i, 0)),
        out_shape=jax.ShapeDtypeStruct((n2, 128), jnp.float32),
    )(hp, d128)


def _tc_layer(p, g, b, W_bd, d128):
    n2 = g.shape[0]
    return pl.pallas_call(
        _layer_body,
        grid=(n2 // _R2,),
        in_specs=[
            pl.BlockSpec((2, _R2, 128), lambda i: (0, i, 0)),
            pl.BlockSpec((_R2, 128), lambda i: (i, 0)),
            pl.BlockSpec((1, 128), lambda i: (0, 0)),
            pl.BlockSpec((128, 128), lambda i: (0, 0)),
            pl.BlockSpec((_R2, 128), lambda i: (i, 0)),
        ],
        out_specs=pl.BlockSpec((_R2, 128), lambda i: (i, 0)),
        out_shape=jax.ShapeDtypeStruct((n2, 128), jnp.float32),
    )(p, g, b, W_bd, d128)


def _tc_final(p, g, b, d128):
    n2 = g.shape[0]
    return pl.pallas_call(
        _final_body,
        grid=(n2 // _R2,),
        in_specs=[
            pl.BlockSpec((2, _R2, 128), lambda i: (0, i, 0)),
            pl.BlockSpec((_R2, 128), lambda i: (i, 0)),
            pl.BlockSpec((1, 128), lambda i: (0, 0)),
            pl.BlockSpec((_R2, 128), lambda i: (i, 0)),
        ],
        out_specs=pl.BlockSpec((_R2, 128), lambda i: (i, 0)),
        out_shape=jax.ShapeDtypeStruct((n2, 128), jnp.float32),
    )(p, g, b, d128)


def _blockdiag(W):
    r, c = W.shape
    return (jnp.zeros((2 * r, 2 * c), jnp.float32)
            .at[:r, :c].set(W).at[r:, c:].set(W))


def kernel(x, edge_index, W1, b1, W2, b2, W3, b3):
    n = x.shape[0]
    e = edge_index.shape[1]
    n_pad = -(-n // 1024) * 1024
    n2 = n_pad // 2
    cpt = -(-e // (_NW * _CHUNK))
    cpt = -(-cpt // 8) * 8
    e_pad = _NW * cpt * _CHUNK

    spare = n_pad - n
    if spare > 0:
        fill = n + jnp.arange(e_pad - e, dtype=jnp.int32) % spare
    else:
        fill = jnp.full((e_pad - e,), n_pad - 1, jnp.int32)
    eip = jnp.concatenate(
        [edge_index.astype(jnp.int32), jnp.broadcast_to(fill, (2, e_pad - e))],
        axis=1).reshape(2, _NW, cpt, _CHUNK)

    xpair = (jnp.zeros((n2, 256), jnp.float32)
             .at[:n // 2].set(x.reshape(n // 2, 256)))
    W1_bd = _blockdiag(W1)
    W2_bd = _blockdiag(W2)
    W3_bd = _blockdiag(jnp.zeros((64, 64), jnp.float32).at[:, :6].set(W3))
    b1_bd = jnp.concatenate([b1, b1]).reshape(1, 128)
    b2_bd = jnp.concatenate([b2, b2]).reshape(1, 128)
    b3p = jnp.zeros((64,), jnp.float32).at[:6].set(b3)
    b3_bd = jnp.concatenate([b3p, b3p]).reshape(1, 128)

    d128 = _sc_degree_dinv(n_pad, cpt)(eip)

    agg = _sc_segment_sum(n_pad, cpt, 64)
    h1p = _tc_h1(xpair, W1_bd)
    g1p = _tc_scale(h1p, d128)
    p1 = agg(g1p.reshape(n_pad, 64), eip)
    g2p = _tc_layer(p1.reshape(2, n2, 128), g1p, b1_bd, W2_bd, d128)
    p2 = agg(g2p.reshape(n_pad, 64), eip)
    g3p = _tc_layer(p2.reshape(2, n2, 128), g2p, b2_bd, W3_bd, d128)
    p3 = agg(g3p.reshape(n_pad, 64), eip)
    outp = _tc_final(p3.reshape(2, n2, 128), g3p, b3_bd, d128)
    ev = outp[:, :6]
    od = outp[:, 64:70]
    return jnp.stack([ev, od], axis=1).reshape(n_pad, 6)[:n]


# speedup vs baseline: 45.4077x; 1.0373x over previous
import functools

import jax
import jax.numpy as jnp
from jax import lax
from jax.experimental import pallas as pl
from jax.experimental.pallas import tpu as pltpu
from jax.experimental.pallas import tpu_sc as plsc

_SC_PARAMS = pltpu.CompilerParams(use_tc_tiling_on_sc=False)

_NC = 2
_NS = 16
_NW = _NC * _NS
_CHUNK = 128
_DEG_W = 16


@functools.lru_cache(maxsize=None)
def _sc_segment_sum(n_pad: int, cpt: int, d: int):
    rpt = n_pad // _NS
    mesh = plsc.VectorSubcoreMesh(core_axis_name="c", subcore_axis_name="s")

    @functools.partial(
        pl.kernel,
        out_type=jax.ShapeDtypeStruct((_NC, n_pad, d), jnp.float32),
        mesh=mesh,
        scratch_types=[
            pltpu.VMEM((cpt, _CHUNK), jnp.int32),
            pltpu.VMEM((cpt, _CHUNK), jnp.int32),
            pltpu.VMEM((8, _CHUNK, d), jnp.float32),
            pltpu.VMEM((64, d), jnp.float32),
            pltpu.VMEM_SHARED((n_pad, d), jnp.float32),
            pltpu.SemaphoreType.DMA((8,)),
            pltpu.SemaphoreType.DMA((8,)),
        ],
        compiler_params=_SC_PARAMS,
    )
    def kern(g_hbm, ei_hbm, out_hbm,
             src_v, dst_v, rows_v, zero_v, acc_sh, sem_g, sem_s):
        cid = lax.axis_index("c")
        sid = lax.axis_index("s")
        w = cid * _NS + sid

        pltpu.sync_copy(ei_hbm.at[0, w], src_v)
        pltpu.sync_copy(ei_hbm.at[1, w], dst_v)

        zvec = jnp.zeros((16,), jnp.float32)

        @pl.loop(0, 64)
        def _(r):
            @pl.loop(0, d, step=16)
            def _(c0):
                zero_v[r, pl.ds(c0, 16)] = zvec

        base = sid * rpt

        @pl.loop(0, rpt, step=64)
        def _(r0):
            pltpu.sync_copy(zero_v, acc_sh.at[pl.ds(base + r0, 64)])

        plsc.subcore_barrier()

        for u in range(4):
            pltpu.async_copy(g_hbm.at[src_v.at[u]], rows_v.at[u], sem_g.at[u])

        @pl.loop(0, cpt, step=8)
        def _(k0):
            for u in range(8):
                k = k0 + u
                bn = (u + 4) % 8
                pltpu.make_async_copy(g_hbm.at[src_v.at[k]], rows_v.at[u],
                                      sem_g.at[u]).wait()
                pltpu.async_copy(rows_v.at[u], acc_sh.at[dst_v.at[k]],
                                 sem_s.at[u], add=True)

                @pl.when(k + 4 < cpt)
                def _():
                    @pl.when(k >= 4)
                    def _():
                        pltpu.make_async_copy(rows_v.at[bn],
                                              acc_sh.at[dst_v.at[k - 4]],
                                              sem_s.at[bn]).wait()
                    pltpu.async_copy(g_hbm.at[src_v.at[k + 4]], rows_v.at[bn],
                                     sem_g.at[bn])

        for u in range(8):
            pltpu.make_async_copy(rows_v.at[u], acc_sh.at[dst_v.at[cpt - 8 + u]],
                                  sem_s.at[u]).wait()

        plsc.subcore_barrier()
        pltpu.sync_copy(acc_sh.at[pl.ds(base, rpt)],
                        out_hbm.at[cid, pl.ds(base, rpt)])

    return kern


def _rsqrt_vec(v):
    i = lax.bitcast_convert_type(v, jnp.int32)
    y = lax.bitcast_convert_type(jnp.int32(0x5F3759DF) - (i >> 1), jnp.float32)
    y = y * (1.5 - 0.5 * v * y * y)
    y = y * (1.5 - 0.5 * v * y * y)
    y = y * (1.5 - 0.5 * v * y * y)
    return y


@functools.lru_cache(maxsize=None)
def _sc_degree_dinv(n_pad: int, cpt: int):
    d = _DEG_W
    rpt = n_pad // _NS
    half = rpt // 2
    mesh = plsc.VectorSubcoreMesh(core_axis_name="c", subcore_axis_name="s")

    @functools.partial(
        pl.kernel,
        out_type=jax.ShapeDtypeStruct((n_pad // 2, 128), jnp.float32),
        mesh=mesh,
        scratch_types=[
            pltpu.VMEM((2 * cpt, _CHUNK), jnp.int32),
            pltpu.VMEM((_CHUNK, d), jnp.float32),
            pltpu.VMEM((64, d), jnp.float32),
            pltpu.VMEM((rpt, d), jnp.float32),
            pltpu.VMEM((half, 128), jnp.float32),
            pltpu.VMEM_SHARED((n_pad, d), jnp.float32),
            pltpu.SemaphoreType.DMA((4,)),
        ],
        compiler_params=_SC_PARAMS,
    )
    def kern(ei_hbm, out_hbm, dst_v, ones_v, zero_v, deg_v, dinv_v,
             acc_sh, sem_s):
        cid = lax.axis_index("c")
        sid = lax.axis_index("s")

        @pl.when(cid == 0)
        def _():
            pltpu.sync_copy(ei_hbm.at[1, 2 * sid], dst_v.at[pl.ds(0, cpt)])
            pltpu.sync_copy(ei_hbm.at[1, 2 * sid + 1], dst_v.at[pl.ds(cpt, cpt)])

            zvec = jnp.zeros((16,), jnp.float32)
            ovec = jnp.ones((16,), jnp.float32)

            @pl.loop(0, _CHUNK)
            def _(r):
                ones_v[r, pl.ds(0, 16)] = ovec

            @pl.loop(0, 64)
            def _(r):
                zero_v[r, pl.ds(0, 16)] = zvec

            base = sid * rpt

            @pl.loop(0, rpt, step=64)
            def _(r0):
                pltpu.sync_copy(zero_v, acc_sh.at[pl.ds(base + r0, 64)])

            plsc.subcore_barrier()

            @pl.loop(0, 2 * cpt, step=4)
            def _(j):
                for u in range(4):
                    k = j + u

                    @pl.when(k >= 4)
                    def _():
                        pltpu.make_async_copy(ones_v, acc_sh.at[dst_v.at[k - 4]],
                                              sem_s.at[u]).wait()

                    pltpu.async_copy(ones_v, acc_sh.at[dst_v.at[k]],
                                     sem_s.at[u], add=True)

            for u in range(4):
                pltpu.make_async_copy(ones_v, acc_sh.at[dst_v.at[2 * cpt - 4 + u]],
                                      sem_s.at[u]).wait()

            plsc.subcore_barrier()

            pltpu.sync_copy(acc_sh.at[pl.ds(base, rpt)], deg_v)

            @pl.loop(0, half)
            def _(r2):
                ye = _rsqrt_vec(deg_v[2 * r2, pl.ds(0, 16)] + 1.0)
                yo = _rsqrt_vec(deg_v[2 * r2 + 1, pl.ds(0, 16)] + 1.0)
                for q in range(4):
                    dinv_v[r2, pl.ds(16 * q, 16)] = ye
                for q in range(4, 8):
                    dinv_v[r2, pl.ds(16 * q, 16)] = yo

            pltpu.sync_copy(dinv_v, out_hbm.at[pl.ds(sid * half, half)])

    return kern


_MM = dict(preferred_element_type=jnp.float32, precision=lax.Precision.HIGHEST)
_R2 = 1024


def _h1_body(x_ref, w_ref, o_ref):
    o_ref[...] = jnp.dot(x_ref[...], w_ref[...], **_MM)


def _scale_body(h_ref, d_ref, g_ref):
    g_ref[...] = d_ref[...] * h_ref[...]


def _layer_body(p_ref, g_ref, b_ref, w_ref, d_ref, o_ref):
    dinv = d_ref[...]
    s = p_ref[0] + p_ref[1] + g_ref[...]
    o = jnp.maximum(dinv * s + b_ref[...], 0.0)
    o_ref[...] = dinv * jnp.dot(o, w_ref[...], **_MM)


def _final_body(p_ref, g_ref, b_ref, d_ref, o_ref):
    o_ref[...] = d_ref[...] * (p_ref[0] + p_ref[1] + g_ref[...]) + b_ref[...]


def _tc_h1(xpair, W1_bd):
    n2 = xpair.shape[0]
    return pl.pallas_call(
        _h1_body,
        grid=(n2 // _R2,),
        in_specs=[
            pl.BlockSpec((_R2, 256), lambda i: (i, 0)),
            pl.BlockSpec((256, 128), lambda i: (0, 0)),
        ],
        out_specs=pl.BlockSpec((_R2, 128), lambda i: (i, 0)),
        out_shape=jax.ShapeDtypeStruct((n2, 128), jnp.float32),
    )(xpair, W1_bd)


def _tc_scale(hp, d128):
    n2 = hp.shape[0]
    return pl.pallas_call(
        _scale_body,
        grid=(n2 // _R2,),
        in_specs=[
            pl.BlockSpec((_R2, 128), lambda i: (i, 0)),
            pl.BlockSpec((_R2, 128), lambda i: (i, 0)),
        ],
        out_specs=pl.BlockSpec((_R2, 128), lambda i: (i, 0)),
        out_shape=jax.ShapeDtypeStruct((n2, 128), jnp.float32),
    )(hp, d128)


def _tc_layer(p, g, b, W_bd, d128):
    n2 = g.shape[0]
    return pl.pallas_call(
        _layer_body,
        grid=(n2 // _R2,),
        in_specs=[
            pl.BlockSpec((2, _R2, 128), lambda i: (0, i, 0)),
            pl.BlockSpec((_R2, 128), lambda i: (i, 0)),
            pl.BlockSpec((1, 128), lambda i: (0, 0)),
            pl.BlockSpec((128, 128), lambda i: (0, 0)),
            pl.BlockSpec((_R2, 128), lambda i: (i, 0)),
        ],
        out_specs=pl.BlockSpec((_R2, 128), lambda i: (i, 0)),
        out_shape=jax.ShapeDtypeStruct((n2, 128), jnp.float32),
    )(p, g, b, W_bd, d128)


def _tc_final(p, g, b, d128):
    n2 = g.shape[0]
    return pl.pallas_call(
        _final_body,
        grid=(n2 // _R2,),
        in_specs=[
            pl.BlockSpec((2, _R2, 128), lambda i: (0, i, 0)),
            pl.BlockSpec((_R2, 128), lambda i: (i, 0)),
            pl.BlockSpec((1, 128), lambda i: (0, 0)),
            pl.BlockSpec((_R2, 128), lambda i: (i, 0)),
        ],
        out_specs=pl.BlockSpec((_R2, 128), lambda i: (i, 0)),
        out_shape=jax.ShapeDtypeStruct((n2, 128), jnp.float32),
    )(p, g, b, d128)


def _blockdiag(W):
    r, c = W.shape
    return (jnp.zeros((2 * r, 2 * c), jnp.float32)
            .at[:r, :c].set(W).at[r:, c:].set(W))


def kernel(x, edge_index, W1, b1, W2, b2, W3, b3):
    n = x.shape[0]
    e = edge_index.shape[1]
    n_pad = -(-n // 1024) * 1024
    n2 = n_pad // 2
    cpt = -(-e // (_NW * _CHUNK))
    cpt = -(-cpt // 8) * 8
    e_pad = _NW * cpt * _CHUNK

    spare = n_pad - n
    if spare > 0:
        fill = n + jnp.arange(e_pad - e, dtype=jnp.int32) % spare
    else:
        fill = jnp.full((e_pad - e,), n_pad - 1, jnp.int32)
    eip = jnp.concatenate(
        [edge_index.astype(jnp.int32), jnp.broadcast_to(fill, (2, e_pad - e))],
        axis=1).reshape(2, _NW, cpt, _CHUNK)

    xpair = (jnp.zeros((n2, 256), jnp.float32)
             .at[:n // 2].set(x.reshape(n // 2, 256)))
    W1_bd = _blockdiag(W1)
    W2_bd = _blockdiag(W2)
    W3_bd = _blockdiag(jnp.zeros((64, 64), jnp.float32).at[:, :6].set(W3))
    b1_bd = jnp.concatenate([b1, b1]).reshape(1, 128)
    b2_bd = jnp.concatenate([b2, b2]).reshape(1, 128)
    b3p = jnp.zeros((64,), jnp.float32).at[:6].set(b3)
    b3_bd = jnp.concatenate([b3p, b3p]).reshape(1, 128)

    d128 = _sc_degree_dinv(n_pad, cpt)(eip)

    agg = _sc_segment_sum(n_pad, cpt, 64)
    h1p = _tc_h1(xpair, W1_bd)
    g1p = _tc_scale(h1p, d128)
    p1 = agg(g1p.reshape(n_pad, 64), eip)
    g2p = _tc_layer(p1.reshape(2, n2, 128), g1p, b1_bd, W2_bd, d128)
    p2 = agg(g2p.reshape(n_pad, 64), eip)
    g3p = _tc_layer(p2.reshape(2, n2, 128), g2p, b2_bd, W3_bd, d128)
    p3 = agg(g3p.reshape(n_pad, 64), eip)
    outp = _tc_final(p3.reshape(2, n2, 128), g3p, b3_bd, d128)
    out12 = jnp.concatenate([outp[:, :6], outp[:, 64:70]], axis=1)
    return out12.reshape(n_pad, 6)[:n]


# speedup vs baseline: 46.7565x; 1.0297x over previous
import functools

import jax
import jax.numpy as jnp
from jax import lax
from jax.experimental import pallas as pl
from jax.experimental.pallas import tpu as pltpu
from jax.experimental.pallas import tpu_sc as plsc

_SC_PARAMS = pltpu.CompilerParams(use_tc_tiling_on_sc=False)

_NC = 2
_NS = 16
_NW = _NC * _NS
_CHUNK = 128
_DEG_W = 16


@functools.lru_cache(maxsize=None)
def _sc_segment_sum(n_pad: int, cpt: int, d: int):
    rpt = n_pad // _NS
    mesh = plsc.VectorSubcoreMesh(core_axis_name="c", subcore_axis_name="s")

    @functools.partial(
        pl.kernel,
        out_type=jax.ShapeDtypeStruct((_NC, n_pad, d), jnp.float32),
        mesh=mesh,
        scratch_types=[
            pltpu.VMEM((cpt, _CHUNK), jnp.int32),
            pltpu.VMEM((cpt, _CHUNK), jnp.int32),
            pltpu.VMEM((8, _CHUNK, d), jnp.float32),
            pltpu.VMEM((64, d), jnp.float32),
            pltpu.VMEM_SHARED((n_pad, d), jnp.float32),
            pltpu.SemaphoreType.DMA((8,)),
        ],
        compiler_params=_SC_PARAMS,
    )
    def kern(g_hbm, ei_hbm, out_hbm,
             src_v, dst_v, rows_v, zero_v, acc_sh, sem_g):
        cid = lax.axis_index("c")
        sid = lax.axis_index("s")
        w = cid * _NS + sid

        pltpu.sync_copy(ei_hbm.at[0, w], src_v)
        pltpu.sync_copy(ei_hbm.at[1, w], dst_v)

        zvec = jnp.zeros((16,), jnp.float32)

        @pl.loop(0, 64)
        def _(r):
            @pl.loop(0, d, step=16)
            def _(c0):
                zero_v[r, pl.ds(c0, 16)] = zvec

        base = sid * rpt

        @pl.loop(0, rpt, step=64)
        def _(r0):
            pltpu.sync_copy(zero_v, acc_sh.at[pl.ds(base + r0, 64)])

        plsc.subcore_barrier()

        for u in range(4):
            pltpu.async_copy(g_hbm.at[src_v.at[u]], rows_v.at[u], sem_g.at[u])

        @pl.loop(0, cpt, step=8)
        def _(k0):
            for u in range(8):
                k = k0 + u
                bn = (u + 4) % 8
                pltpu.make_async_copy(g_hbm.at[src_v.at[k]], rows_v.at[u],
                                      sem_g.at[u]).wait()

                @pl.when(k + 4 < cpt)
                def _():
                    pltpu.async_copy(g_hbm.at[src_v.at[k + 4]], rows_v.at[bn],
                                     sem_g.at[bn])

                pltpu.sync_copy(rows_v.at[u], acc_sh.at[dst_v.at[k]], add=True)

        plsc.subcore_barrier()
        pltpu.sync_copy(acc_sh.at[pl.ds(base, rpt)],
                        out_hbm.at[cid, pl.ds(base, rpt)])

    return kern


def _rsqrt_vec(v):
    i = lax.bitcast_convert_type(v, jnp.int32)
    y = lax.bitcast_convert_type(jnp.int32(0x5F3759DF) - (i >> 1), jnp.float32)
    y = y * (1.5 - 0.5 * v * y * y)
    y = y * (1.5 - 0.5 * v * y * y)
    y = y * (1.5 - 0.5 * v * y * y)
    return y


@functools.lru_cache(maxsize=None)
def _sc_degree_dinv(n_pad: int, cpt: int):
    d = _DEG_W
    rpt = n_pad // _NS
    half = rpt // 2
    mesh = plsc.VectorSubcoreMesh(core_axis_name="c", subcore_axis_name="s")

    @functools.partial(
        pl.kernel,
        out_type=jax.ShapeDtypeStruct((n_pad // 2, 128), jnp.float32),
        mesh=mesh,
        scratch_types=[
            pltpu.VMEM((2 * cpt, _CHUNK), jnp.int32),
            pltpu.VMEM((_CHUNK, d), jnp.float32),
            pltpu.VMEM((64, d), jnp.float32),
            pltpu.VMEM((rpt, d), jnp.float32),
            pltpu.VMEM((half, 128), jnp.float32),
            pltpu.VMEM_SHARED((n_pad, d), jnp.float32),
        ],
        compiler_params=_SC_PARAMS,
    )
    def kern(ei_hbm, out_hbm, dst_v, ones_v, zero_v, deg_v, dinv_v,
             acc_sh):
        cid = lax.axis_index("c")
        sid = lax.axis_index("s")

        @pl.when(cid == 0)
        def _():
            pltpu.sync_copy(ei_hbm.at[1, 2 * sid], dst_v.at[pl.ds(0, cpt)])
            pltpu.sync_copy(ei_hbm.at[1, 2 * sid + 1], dst_v.at[pl.ds(cpt, cpt)])

            zvec = jnp.zeros((16,), jnp.float32)
            ovec = jnp.ones((16,), jnp.float32)

            @pl.loop(0, _CHUNK)
            def _(r):
                ones_v[r, pl.ds(0, 16)] = ovec

            @pl.loop(0, 64)
            def _(r):
                zero_v[r, pl.ds(0, 16)] = zvec

            base = sid * rpt

            @pl.loop(0, rpt, step=64)
            def _(r0):
                pltpu.sync_copy(zero_v, acc_sh.at[pl.ds(base + r0, 64)])

            plsc.subcore_barrier()

            @pl.loop(0, 2 * cpt)
            def _(k):
                pltpu.sync_copy(ones_v, acc_sh.at[dst_v.at[k]], add=True)

            plsc.subcore_barrier()

            pltpu.sync_copy(acc_sh.at[pl.ds(base, rpt)], deg_v)

            @pl.loop(0, half)
            def _(r2):
                ye = _rsqrt_vec(deg_v[2 * r2, pl.ds(0, 16)] + 1.0)
                yo = _rsqrt_vec(deg_v[2 * r2 + 1, pl.ds(0, 16)] + 1.0)
                for q in range(4):
                    dinv_v[r2, pl.ds(16 * q, 16)] = ye
                for q in range(4, 8):
                    dinv_v[r2, pl.ds(16 * q, 16)] = yo

            pltpu.sync_copy(dinv_v, out_hbm.at[pl.ds(sid * half, half)])

    return kern


_MM = dict(preferred_element_type=jnp.float32, precision=lax.Precision.HIGHEST)
_R2 = 1024


def _h1_body(x_ref, w_ref, o_ref):
    o_ref[...] = jnp.dot(x_ref[...], w_ref[...], **_MM)


def _scale_body(h_ref, d_ref, g_ref):
    g_ref[...] = d_ref[...] * h_ref[...]


def _layer_body(p_ref, g_ref, b_ref, w_ref, d_ref, o_ref):
    dinv = d_ref[...]
    s = p_ref[0] + p_ref[1] + g_ref[...]
    o = jnp.maximum(dinv * s + b_ref[...], 0.0)
    o_ref[...] = dinv * jnp.dot(o, w_ref[...], **_MM)


def _final_body(p_ref, g_ref, b_ref, d_ref, o_ref):
    o_ref[...] = d_ref[...] * (p_ref[0] + p_ref[1] + g_ref[...]) + b_ref[...]


def _tc_h1(xpair, W1_bd):
    n2 = xpair.shape[0]
    return pl.pallas_call(
        _h1_body,
        grid=(n2 // _R2,),
        in_specs=[
            pl.BlockSpec((_R2, 256), lambda i: (i, 0)),
            pl.BlockSpec((256, 128), lambda i: (0, 0)),
        ],
        out_specs=pl.BlockSpec((_R2, 128), lambda i: (i, 0)),
        out_shape=jax.ShapeDtypeStruct((n2, 128), jnp.float32),
    )(xpair, W1_bd)


def _tc_scale(hp, d128):
    n2 = hp.shape[0]
    return pl.pallas_call(
        _scale_body,
        grid=(n2 // _R2,),
        in_specs=[
            pl.BlockSpec((_R2, 128), lambda i: (i, 0)),
            pl.BlockSpec((_R2, 128), lambda i: (i, 0)),
        ],
        out_specs=pl.BlockSpec((_R2, 128), lambda i: (i, 0)),
        out_shape=jax.ShapeDtypeStruct((n2, 128), jnp.float32),
    )(hp, d128)


def _tc_layer(p, g, b, W_bd, d128):
    n2 = g.shape[0]
    return pl.pallas_call(
        _layer_body,
        grid=(n2 // _R2,),
        in_specs=[
            pl.BlockSpec((2, _R2, 128), lambda i: (0, i, 0)),
            pl.BlockSpec((_R2, 128), lambda i: (i, 0)),
            pl.BlockSpec((1, 128), lambda i: (0, 0)),
            pl.BlockSpec((128, 128), lambda i: (0, 0)),
            pl.BlockSpec((_R2, 128), lambda i: (i, 0)),
        ],
        out_specs=pl.BlockSpec((_R2, 128), lambda i: (i, 0)),
        out_shape=jax.ShapeDtypeStruct((n2, 128), jnp.float32),
    )(p, g, b, W_bd, d128)


def _tc_final(p, g, b, d128):
    n2 = g.shape[0]
    return pl.pallas_call(
        _final_body,
        grid=(n2 // _R2,),
        in_specs=[
            pl.BlockSpec((2, _R2, 128), lambda i: (0, i, 0)),
            pl.BlockSpec((_R2, 128), lambda i: (i, 0)),
            pl.BlockSpec((1, 128), lambda i: (0, 0)),
            pl.BlockSpec((_R2, 128), lambda i: (i, 0)),
        ],
        out_specs=pl.BlockSpec((_R2, 128), lambda i: (i, 0)),
        out_shape=jax.ShapeDtypeStruct((n2, 128), jnp.float32),
    )(p, g, b, d128)


def _blockdiag(W):
    r, c = W.shape
    return (jnp.zeros((2 * r, 2 * c), jnp.float32)
            .at[:r, :c].set(W).at[r:, c:].set(W))


def kernel(x, edge_index, W1, b1, W2, b2, W3, b3):
    n = x.shape[0]
    e = edge_index.shape[1]
    n_pad = -(-n // 1024) * 1024
    n2 = n_pad // 2
    cpt = -(-e // (_NW * _CHUNK))
    cpt = -(-cpt // 8) * 8
    e_pad = _NW * cpt * _CHUNK

    spare = n_pad - n
    if spare > 0:
        fill = n + jnp.arange(e_pad - e, dtype=jnp.int32) % spare
    else:
        fill = jnp.full((e_pad - e,), n_pad - 1, jnp.int32)
    eip = jnp.concatenate(
        [edge_index.astype(jnp.int32), jnp.broadcast_to(fill, (2, e_pad - e))],
        axis=1).reshape(2, _NW, cpt, _CHUNK)

    xpair = (jnp.zeros((n2, 256), jnp.float32)
             .at[:n // 2].set(x.reshape(n // 2, 256)))
    W1_bd = _blockdiag(W1)
    W2_bd = _blockdiag(W2)
    W3_bd = _blockdiag(jnp.zeros((64, 64), jnp.float32).at[:, :6].set(W3))
    b1_bd = jnp.concatenate([b1, b1]).reshape(1, 128)
    b2_bd = jnp.concatenate([b2, b2]).reshape(1, 128)
    b3p = jnp.zeros((64,), jnp.float32).at[:6].set(b3)
    b3_bd = jnp.concatenate([b3p, b3p]).reshape(1, 128)

    d128 = _sc_degree_dinv(n_pad, cpt)(eip)

    agg = _sc_segment_sum(n_pad, cpt, 64)
    h1p = _tc_h1(xpair, W1_bd)
    g1p = _tc_scale(h1p, d128)
    p1 = agg(g1p.reshape(n_pad, 64), eip)
    g2p = _tc_layer(p1.reshape(2, n2, 128), g1p, b1_bd, W2_bd, d128)
    p2 = agg(g2p.reshape(n_pad, 64), eip)
    g3p = _tc_layer(p2.reshape(2, n2, 128), g2p, b2_bd, W3_bd, d128)
    p3 = agg(g3p.reshape(n_pad, 64), eip)
    outp = _tc_final(p3.reshape(2, n2, 128), g3p, b3_bd, d128)
    out12 = jnp.concatenate([outp[:, :6], outp[:, 64:70]], axis=1)
    return out12.reshape(n_pad, 6)[:n]
